# baseline (speedup 1.0000x reference)
import jax
import jax.numpy as jnp
from jax.experimental import pallas as pl


def _pick_block(n, candidates):
    for c in candidates:
        if n % c == 0:
            return c
    return n


def _support_body(x_ref, w_ref, out_ref):
    out_ref[...] = jax.lax.dot(
        x_ref[...].astype(jnp.bfloat16),
        w_ref[...].astype(jnp.bfloat16),
        preferred_element_type=jnp.float32,
    ).astype(jnp.bfloat16)


def _spmm_body(adj_ref, s_ref, b_ref, out_ref):
    acc = jax.lax.dot(
        adj_ref[...].astype(jnp.bfloat16),
        s_ref[...],
        preferred_element_type=jnp.float32,
    )
    out_ref[...] = acc + b_ref[...]


def kernel(input, adj, weight, bias):
    n, din = input.shape
    dout = weight.shape[1]

    bm1 = _pick_block(n, (2000, 1000, 500, 250, 200, 100, 8))
    support = pl.pallas_call(
        _support_body,
        grid=(n // bm1,),
        in_specs=[
            pl.BlockSpec((bm1, din), lambda i: (i, 0)),
            pl.BlockSpec((din, dout), lambda i: (0, 0)),
        ],
        out_specs=pl.BlockSpec((bm1, dout), lambda i: (i, 0)),
        out_shape=jax.ShapeDtypeStruct((n, dout), jnp.bfloat16),
    )(input, weight)

    bm = _pick_block(n, (200, 100, 8))
    out = pl.pallas_call(
        _spmm_body,
        grid=(n // bm,),
        in_specs=[
            pl.BlockSpec((bm, n), lambda i: (i, 0)),
            pl.BlockSpec((n, dout), lambda i: (0, 0)),
            pl.BlockSpec((1, dout), lambda i: (0, 0)),
        ],
        out_specs=pl.BlockSpec((bm, dout), lambda i: (i, 0)),
        out_shape=jax.ShapeDtypeStruct((n, dout), jnp.float32),
    )(adj, support, bias)
    return out


# speedup vs baseline: 1.0562x; 1.0562x over previous
import jax
import jax.numpy as jnp
from jax.experimental import pallas as pl
from jax.experimental.pallas import tpu as pltpu


def _pick_block(n, candidates):
    for c in candidates:
        if n % c == 0:
            return c
    return n


def _fused_body(x_ref, w_ref, b_ref, adj_ref, out_ref, s_ref):
    @pl.when(pl.program_id(0) == 0)
    def _compute_support():
        s_ref[...] = jax.lax.dot(
            x_ref[...].astype(jnp.bfloat16),
            w_ref[...].astype(jnp.bfloat16),
            preferred_element_type=jnp.float32,
        ).astype(jnp.bfloat16)

    acc = jax.lax.dot(
        adj_ref[...].astype(jnp.bfloat16),
        s_ref[...],
        preferred_element_type=jnp.float32,
    )
    out_ref[...] = acc + b_ref[...]


def kernel(input, adj, weight, bias):
    n, din = input.shape
    dout = weight.shape[1]

    bm = _pick_block(n, (400, 200, 100, 8))
    out = pl.pallas_call(
        _fused_body,
        grid=(n // bm,),
        in_specs=[
            pl.BlockSpec((n, din), lambda i: (0, 0)),
            pl.BlockSpec((din, dout), lambda i: (0, 0)),
            pl.BlockSpec((1, dout), lambda i: (0, 0)),
            pl.BlockSpec((bm, n), lambda i: (i, 0)),
        ],
        out_specs=pl.BlockSpec((bm, dout), lambda i: (i, 0)),
        out_shape=jax.ShapeDtypeStruct((n, dout), jnp.float32),
        scratch_shapes=[pltpu.VMEM((n, dout), jnp.bfloat16)],
    )(input, weight, bias, adj)
    return out


# speedup vs baseline: 1.0629x; 1.0063x over previous
import jax
import jax.numpy as jnp
from jax.experimental import pallas as pl
from jax.experimental.pallas import tpu as pltpu


def _pick_block(n, candidates):
    for c in candidates:
        if n % c == 0:
            return c
    return n


def _fused_body(x_ref, w_ref, b_ref, adj_ref, out_ref, xb_ref):
    @pl.when(pl.program_id(0) == 0)
    def _cast_input():
        xb_ref[...] = x_ref[...].astype(jnp.bfloat16)

    t = jax.lax.dot(
        adj_ref[...].astype(jnp.bfloat16),
        xb_ref[...],
        preferred_element_type=jnp.float32,
    )
    out_ref[...] = (
        jax.lax.dot(
            t.astype(jnp.bfloat16),
            w_ref[...].astype(jnp.bfloat16),
            preferred_element_type=jnp.float32,
        )
        + b_ref[...]
    )


def kernel(input, adj, weight, bias):
    n, din = input.shape
    dout = weight.shape[1]

    bm = _pick_block(n, (400, 200, 100, 8))
    out = pl.pallas_call(
        _fused_body,
        grid=(n // bm,),
        in_specs=[
            pl.BlockSpec((n, din), lambda i: (0, 0)),
            pl.BlockSpec((din, dout), lambda i: (0, 0)),
            pl.BlockSpec((1, dout), lambda i: (0, 0)),
            pl.BlockSpec((bm, n), lambda i: (i, 0)),
        ],
        out_specs=pl.BlockSpec((bm, dout), lambda i: (i, 0)),
        out_shape=jax.ShapeDtypeStruct((n, dout), jnp.float32),
        scratch_shapes=[pltpu.VMEM((n, din), jnp.bfloat16)],
    )(input, weight, bias, adj)
    return out
